# baseline (speedup 1.0000x reference)
import jax
import jax.numpy as jnp
from jax.experimental import pallas as pl

_K = 8192
_D = 256
_TILE_M = 1024
_B1 = 2736
_B2 = 5472
_A1 = 2688
_A2 = 5376
_BIG = 16384.0


def _rowmin(x):
    return jnp.min(x, axis=1, keepdims=True)


def _first_idx(d, m, lane0):
    fiota = lane0 + jax.lax.broadcasted_iota(
        jnp.int32, d.shape, 1).astype(jnp.float32)
    return _rowmin(jnp.where(d == m, fiota, _BIG))


def _vq_kernel(xbf_ref, wbf_ref, zsq_ref, wsq_ref, out_ref):
    ncross = jax.lax.dot_general(
        xbf_ref[...], wbf_ref[...], (((1,), (0,)), ((), ())),
        preferred_element_type=jnp.float32)
    dists = (zsq_ref[...] + wsq_ref[...]) + ncross

    inf = jnp.float32(jnp.inf)
    lane = jax.lax.broadcasted_iota(jnp.int32, (_TILE_M, 128), 1)
    dA = dists[:, :_A1]
    dB = dists[:, _A1 + 128:_A2]
    dC = dists[:, _A2 + 128:]
    s1 = dists[:, _A1:_A1 + 128]
    s2 = dists[:, _A2:_A2 + 128]
    s1lo = jnp.where(lane < _B1 - _A1, s1, inf)
    s1hi = jnp.where(lane >= _B1 - _A1, s1, inf)
    s2lo = jnp.where(lane < _B2 - _A2, s2, inf)
    s2hi = jnp.where(lane >= _B2 - _A2, s2, inf)

    m1 = jnp.minimum(_rowmin(dA), _rowmin(s1lo))
    m2 = jnp.minimum(jnp.minimum(_rowmin(s1hi), _rowmin(dB)), _rowmin(s2lo))
    m3 = jnp.minimum(_rowmin(s2hi), _rowmin(dC))

    f1 = jnp.minimum(_first_idx(dA, m1, 0.0), _first_idx(s1lo, m1, float(_A1)))
    f2 = jnp.minimum(
        jnp.minimum(_first_idx(s1hi, m2, float(_A1)),
                    _first_idx(dB, m2, float(_A1 + 128))),
        _first_idx(s2lo, m2, float(_A2)))
    f3 = jnp.minimum(_first_idx(s2hi, m3, float(_A2)),
                     _first_idx(dC, m3, float(_A2 + 128)))

    vb1 = m1.astype(jnp.bfloat16).astype(jnp.float32)
    take2 = m2 < vb1
    v2 = jnp.where(take2, m2, vb1)
    vb2 = v2.astype(jnp.bfloat16).astype(jnp.float32)
    take3 = m3 < vb2
    fstar = jnp.where(take3, f3, jnp.where(take2, f2, f1))
    idx = fstar.astype(jnp.int32)
    out_ref[...] = idx.reshape(1, 1, _TILE_M)


def kernel(z_e_x, embedding_weight):
    b, t, d = z_e_x.shape
    n = b * t
    flat = z_e_x.reshape(n, d)
    zsq = jnp.sum(flat * flat, axis=1, keepdims=True)
    wsq = jnp.sum(embedding_weight * embedding_weight, axis=1)[None]
    xbf = (flat * (-2.0)).astype(jnp.bfloat16)
    wbf = embedding_weight.astype(jnp.bfloat16).T
    grid = n // _TILE_M
    out = pl.pallas_call(
        _vq_kernel,
        grid=(grid,),
        in_specs=[
            pl.BlockSpec((_TILE_M, d), lambda i: (i, 0)),
            pl.BlockSpec((d, _K), lambda i: (0, 0)),
            pl.BlockSpec((_TILE_M, 1), lambda i: (i, 0)),
            pl.BlockSpec((1, _K), lambda i: (0, 0)),
        ],
        out_specs=pl.BlockSpec((1, 1, _TILE_M), lambda i: (i, 0, 0)),
        out_shape=jax.ShapeDtypeStruct((grid, 1, _TILE_M), jnp.int32),
    )(xbf, wbf, zsq, wsq)
    return out.reshape(b, t)


# speedup vs baseline: 1.0338x; 1.0338x over previous
import jax
import jax.numpy as jnp
from jax.experimental import pallas as pl

_K = 8192
_D = 256
_TILE_M = 1024
_B1 = 2736
_B2 = 5472
_A1 = 2688
_A2 = 5376
_BIG = 16384.0


def _rowmin(x):
    return jnp.min(x, axis=1, keepdims=True)


def _first_idx(d, m, lane0):
    fiota = lane0 + jax.lax.broadcasted_iota(
        jnp.int32, d.shape, 1).astype(jnp.float32)
    return _rowmin(jnp.where(d == m, fiota, _BIG))


def _vq_kernel(xbf_ref, wbf_ref, zsq_ref, wsq_ref, out_ref):
    ncross = jax.lax.dot_general(
        xbf_ref[...], wbf_ref[...], (((1,), (1,)), ((), ())),
        preferred_element_type=jnp.float32)
    dists = (zsq_ref[...] + wsq_ref[...]) + ncross

    inf = jnp.float32(jnp.inf)
    lane = jax.lax.broadcasted_iota(jnp.int32, (_TILE_M, 128), 1)
    dA = dists[:, :_A1]
    dB = dists[:, _A1 + 128:_A2]
    dC = dists[:, _A2 + 128:]
    s1 = dists[:, _A1:_A1 + 128]
    s2 = dists[:, _A2:_A2 + 128]
    s1lo = jnp.where(lane < _B1 - _A1, s1, inf)
    s1hi = jnp.where(lane >= _B1 - _A1, s1, inf)
    s2lo = jnp.where(lane < _B2 - _A2, s2, inf)
    s2hi = jnp.where(lane >= _B2 - _A2, s2, inf)

    m1 = jnp.minimum(_rowmin(dA), _rowmin(s1lo))
    m2 = jnp.minimum(jnp.minimum(_rowmin(s1hi), _rowmin(dB)), _rowmin(s2lo))
    m3 = jnp.minimum(_rowmin(s2hi), _rowmin(dC))

    f1 = jnp.minimum(_first_idx(dA, m1, 0.0), _first_idx(s1lo, m1, float(_A1)))
    f2 = jnp.minimum(
        jnp.minimum(_first_idx(s1hi, m2, float(_A1)),
                    _first_idx(dB, m2, float(_A1 + 128))),
        _first_idx(s2lo, m2, float(_A2)))
    f3 = jnp.minimum(_first_idx(s2hi, m3, float(_A2)),
                     _first_idx(dC, m3, float(_A2 + 128)))

    vb1 = m1.astype(jnp.bfloat16).astype(jnp.float32)
    take2 = m2 < vb1
    v2 = jnp.where(take2, m2, vb1)
    vb2 = v2.astype(jnp.bfloat16).astype(jnp.float32)
    take3 = m3 < vb2
    fstar = jnp.where(take3, f3, jnp.where(take2, f2, f1))
    idx = fstar.astype(jnp.int32)
    out_ref[...] = idx.reshape(1, 1, _TILE_M)


def kernel(z_e_x, embedding_weight):
    b, t, d = z_e_x.shape
    n = b * t
    flat = z_e_x.reshape(n, d)
    zsq = jnp.sum(flat * flat, axis=1, keepdims=True)
    wsq = jnp.sum(embedding_weight * embedding_weight, axis=1)[None]
    xbf = (flat * (-2.0)).astype(jnp.bfloat16)
    wbf = embedding_weight.astype(jnp.bfloat16)
    grid = n // _TILE_M
    out = pl.pallas_call(
        _vq_kernel,
        grid=(grid,),
        in_specs=[
            pl.BlockSpec((_TILE_M, d), lambda i: (i, 0)),
            pl.BlockSpec((_K, d), lambda i: (0, 0)),
            pl.BlockSpec((_TILE_M, 1), lambda i: (i, 0)),
            pl.BlockSpec((1, _K), lambda i: (0, 0)),
        ],
        out_specs=pl.BlockSpec((1, 1, _TILE_M), lambda i: (i, 0, 0)),
        out_shape=jax.ShapeDtypeStruct((grid, 1, _TILE_M), jnp.int32),
    )(xbf, wbf, zsq, wsq)
    return out.reshape(b, t)


# speedup vs baseline: 1.0495x; 1.0152x over previous
import jax
import jax.numpy as jnp
from jax.experimental import pallas as pl

_K = 8192
_D = 256
_TILE_M = 1024
_B1 = 2736
_B2 = 5472
_A1 = 2688
_A2 = 5376
_BIG = 16384.0


def _rowmin(x):
    return jnp.min(x, axis=1, keepdims=True)


def _first_idx(d, m, lane0):
    fiota = lane0 + jax.lax.broadcasted_iota(
        jnp.int32, d.shape, 1).astype(jnp.float32)
    return _rowmin(jnp.where(d == m, fiota, _BIG))


def _vq_kernel(x_ref, wbf_ref, zsq_ref, wsq_ref, out_ref):
    xbf = (x_ref[...] * (-2.0)).astype(jnp.bfloat16)
    ncross = jax.lax.dot_general(
        xbf, wbf_ref[...], (((1,), (1,)), ((), ())),
        preferred_element_type=jnp.float32)
    dists = (zsq_ref[...] + wsq_ref[...]) + ncross

    inf = jnp.float32(jnp.inf)
    lane = jax.lax.broadcasted_iota(jnp.int32, (_TILE_M, 128), 1)
    dA = dists[:, :_A1]
    dB = dists[:, _A1 + 128:_A2]
    dC = dists[:, _A2 + 128:]
    s1 = dists[:, _A1:_A1 + 128]
    s2 = dists[:, _A2:_A2 + 128]
    s1lo = jnp.where(lane < _B1 - _A1, s1, inf)
    s1hi = jnp.where(lane >= _B1 - _A1, s1, inf)
    s2lo = jnp.where(lane < _B2 - _A2, s2, inf)
    s2hi = jnp.where(lane >= _B2 - _A2, s2, inf)

    m1 = jnp.minimum(_rowmin(dA), _rowmin(s1lo))
    m2 = jnp.minimum(jnp.minimum(_rowmin(s1hi), _rowmin(dB)), _rowmin(s2lo))
    m3 = jnp.minimum(_rowmin(s2hi), _rowmin(dC))

    f1 = jnp.minimum(_first_idx(dA, m1, 0.0), _first_idx(s1lo, m1, float(_A1)))
    f2 = jnp.minimum(
        jnp.minimum(_first_idx(s1hi, m2, float(_A1)),
                    _first_idx(dB, m2, float(_A1 + 128))),
        _first_idx(s2lo, m2, float(_A2)))
    f3 = jnp.minimum(_first_idx(s2hi, m3, float(_A2)),
                     _first_idx(dC, m3, float(_A2 + 128)))

    vb1 = m1.astype(jnp.bfloat16).astype(jnp.float32)
    take2 = m2 < vb1
    v2 = jnp.where(take2, m2, vb1)
    vb2 = v2.astype(jnp.bfloat16).astype(jnp.float32)
    take3 = m3 < vb2
    fstar = jnp.where(take3, f3, jnp.where(take2, f2, f1))
    idx = fstar.astype(jnp.int32)
    out_ref[...] = idx.reshape(1, 1, _TILE_M)


def kernel(z_e_x, embedding_weight):
    b, t, d = z_e_x.shape
    n = b * t
    flat = z_e_x.reshape(n, d)
    zsq = jnp.sum(flat * flat, axis=1, keepdims=True)
    wsq = jnp.sum(embedding_weight * embedding_weight, axis=1)[None]
    wbf = embedding_weight.astype(jnp.bfloat16)
    grid = n // _TILE_M
    out = pl.pallas_call(
        _vq_kernel,
        grid=(grid,),
        in_specs=[
            pl.BlockSpec((_TILE_M, d), lambda i: (i, 0)),
            pl.BlockSpec((_K, d), lambda i: (0, 0)),
            pl.BlockSpec((_TILE_M, 1), lambda i: (i, 0)),
            pl.BlockSpec((1, _K), lambda i: (0, 0)),
        ],
        out_specs=pl.BlockSpec((1, 1, _TILE_M), lambda i: (i, 0, 0)),
        out_shape=jax.ShapeDtypeStruct((grid, 1, _TILE_M), jnp.int32),
    )(flat, wbf, zsq, wsq)
    return out.reshape(b, t)


# speedup vs baseline: 1.1291x; 1.0759x over previous
import jax
import jax.numpy as jnp
from jax.experimental import pallas as pl

_K = 8192
_D = 256
_TILE_M = 1024
_B1 = 2736
_B2 = 5472
_A1 = 2688
_A2 = 5376
_BIG = 16384.0


def _rowmin(x):
    return jnp.min(x, axis=1, keepdims=True)


def _first_idx(d, m, lane0):
    fiota = lane0 + jax.lax.broadcasted_iota(
        jnp.int32, d.shape, 1).astype(jnp.float32)
    return _rowmin(jnp.where(d == m, fiota, _BIG))


def _vq_kernel(x_ref, wbf_ref, wsq_ref, out_ref):
    x = x_ref[...]
    xbf = (x * (-2.0)).astype(jnp.bfloat16)
    ncross = jax.lax.dot_general(
        xbf, wbf_ref[...], (((1,), (1,)), ((), ())),
        preferred_element_type=jnp.float32)
    zsq = jnp.sum(x * x, axis=1, keepdims=True)
    dists = (zsq + wsq_ref[...]) + ncross

    inf = jnp.float32(jnp.inf)
    lane = jax.lax.broadcasted_iota(jnp.int32, (_TILE_M, 128), 1)
    dA = dists[:, :_A1]
    dB = dists[:, _A1 + 128:_A2]
    dC = dists[:, _A2 + 128:]
    s1 = dists[:, _A1:_A1 + 128]
    s2 = dists[:, _A2:_A2 + 128]
    s1lo = jnp.where(lane < _B1 - _A1, s1, inf)
    s1hi = jnp.where(lane >= _B1 - _A1, s1, inf)
    s2lo = jnp.where(lane < _B2 - _A2, s2, inf)
    s2hi = jnp.where(lane >= _B2 - _A2, s2, inf)

    m1 = jnp.minimum(_rowmin(dA), _rowmin(s1lo))
    m2 = jnp.minimum(jnp.minimum(_rowmin(s1hi), _rowmin(dB)), _rowmin(s2lo))
    m3 = jnp.minimum(_rowmin(s2hi), _rowmin(dC))

    f1 = jnp.minimum(_first_idx(dA, m1, 0.0), _first_idx(s1lo, m1, float(_A1)))
    f2 = jnp.minimum(
        jnp.minimum(_first_idx(s1hi, m2, float(_A1)),
                    _first_idx(dB, m2, float(_A1 + 128))),
        _first_idx(s2lo, m2, float(_A2)))
    f3 = jnp.minimum(_first_idx(s2hi, m3, float(_A2)),
                     _first_idx(dC, m3, float(_A2 + 128)))

    vb1 = m1.astype(jnp.bfloat16).astype(jnp.float32)
    take2 = m2 < vb1
    v2 = jnp.where(take2, m2, vb1)
    vb2 = v2.astype(jnp.bfloat16).astype(jnp.float32)
    take3 = m3 < vb2
    fstar = jnp.where(take3, f3, jnp.where(take2, f2, f1))
    idx = fstar.astype(jnp.int32)
    out_ref[...] = idx.reshape(1, 1, _TILE_M)


def kernel(z_e_x, embedding_weight):
    b, t, d = z_e_x.shape
    n = b * t
    flat = z_e_x.reshape(n, d)
    wsq = jnp.sum(embedding_weight * embedding_weight, axis=1)[None]
    wbf = embedding_weight.astype(jnp.bfloat16)
    grid = n // _TILE_M
    out = pl.pallas_call(
        _vq_kernel,
        grid=(grid,),
        in_specs=[
            pl.BlockSpec((_TILE_M, d), lambda i: (i, 0)),
            pl.BlockSpec((_K, d), lambda i: (0, 0)),
            pl.BlockSpec((1, _K), lambda i: (0, 0)),
        ],
        out_specs=pl.BlockSpec((1, 1, _TILE_M), lambda i: (i, 0, 0)),
        out_shape=jax.ShapeDtypeStruct((grid, 1, _TILE_M), jnp.int32),
    )(flat, wbf, wsq)
    return out.reshape(b, t)
